# baseline (speedup 1.0000x reference)
import functools

import jax
import jax.numpy as jnp
from jax import lax
from jax.experimental import pallas as pl
from jax.experimental.pallas import tpu as pltpu
from jax.experimental.pallas import tpu_sc as plsc

B = 16384
D = 64
NC = 2
NS = 16
NW = NC * NS
BPW = B // NW
CHUNK = 128
NCH = BPW // CHUNK
NT = 3
L = 16

_mesh = plsc.VectorSubcoreMesh(core_axis_name="c", subcore_axis_name="s")


@functools.partial(
    pl.kernel,
    mesh=_mesh,
    out_type=jax.ShapeDtypeStruct((NT * B, D), jnp.float32),
    scratch_types=[
        pltpu.VMEM((NT * BPW,), jnp.int32),
        pltpu.VMEM((NT * BPW,), jnp.int32),
        pltpu.VMEM((NT * BPW,), jnp.int32),
        pltpu.VMEM((CHUNK, 2 * D), jnp.float32),
        pltpu.VMEM((CHUNK, D), jnp.float32),
        pltpu.SemaphoreType.DMA,
    ],
)
def _gather3(src_hbm, dst_hbm, rels_hbm, node_hbm, rel_hbm, out_hbm,
             idx_v, qid_v, half_v, pairs_v, rows_v, sem):
    wid = lax.axis_index("s") * NC + lax.axis_index("c")
    base = wid * BPW
    idx_srcs = (src_hbm, dst_hbm, rels_hbm)
    tables = (node_hbm, node_hbm, rel_hbm)

    for t in range(NT):
        pltpu.sync_copy(idx_srcs[t].at[pl.ds(base, BPW)],
                        idx_v.at[pl.ds(t * BPW, BPW)])

    for k in range(NT * BPW // L):
        v = idx_v[pl.ds(k * L, L)]
        qid_v[pl.ds(k * L, L)] = lax.shift_right_logical(v, 1)
        half_v[pl.ds(k * L, L)] = (v & 1) * D

    for t in range(NT):
        table = tables[t]

        def chunk_body(c, _, table=table, t=t):
            off = t * BPW + c * CHUNK
            pltpu.async_copy(
                table.at[qid_v.at[pl.ds(off, CHUNK)]], pairs_v, sem
            ).wait()
            for g in range(CHUNK // L):
                h16 = half_v[pl.ds(off + g * L, L)]
                for j in range(L):
                    h = h16[j]
                    row = g * L + j
                    for l in range(D // L):
                        rows_v[row, pl.ds(l * L, L)] = (
                            pairs_v[row, pl.ds(h + l * L, L)])
            pltpu.sync_copy(
                rows_v,
                out_hbm.at[pl.ds(t * B + base + c * CHUNK, CHUNK)])
            return ()

        lax.fori_loop(0, NCH, chunk_body, (), unroll=False)


def kernel(src, dst, rels, node_table, rel_table):
    node2 = node_table.reshape(500000, 2 * D)
    rel2 = rel_table.reshape(500, 2 * D)
    out = _gather3(src.astype(jnp.int32), dst.astype(jnp.int32),
                   rels.astype(jnp.int32), node2, rel2)
    return out.reshape(NT, B, D)


# speedup vs baseline: 2.5120x; 2.5120x over previous
import functools

import jax
import jax.numpy as jnp
from jax import lax
from jax.experimental import pallas as pl
from jax.experimental.pallas import tpu as pltpu
from jax.experimental.pallas import tpu_sc as plsc

B = 16384
D = 64
NC = 2
NS = 16
NW = NC * NS
BPW = B // NW
NT = 3
L = 16
HB = BPW // 2
NGH = HB // L

_mesh = plsc.VectorSubcoreMesh(core_axis_name="c", subcore_axis_name="s")


@functools.partial(
    pl.kernel,
    mesh=_mesh,
    out_type=jax.ShapeDtypeStruct((NT * B, D), jnp.float32),
    scratch_types=[
        pltpu.VMEM((NT * BPW,), jnp.int32),
        pltpu.VMEM((HB, D), jnp.float32),
        pltpu.VMEM((HB, D), jnp.float32),
        pltpu.SemaphoreType.DMA,
        pltpu.SemaphoreType.DMA,
    ],
)
def _gather3(src_hbm, dst_hbm, rels_hbm, node_hbm, rel_hbm, out_hbm,
             idx_v, rows_a, rows_b, sem_a, sem_b):
    wid = lax.axis_index("s") * NC + lax.axis_index("c")
    base = wid * BPW
    idx_srcs = (src_hbm, dst_hbm, rels_hbm)
    tables = (node_hbm, node_hbm, rel_hbm)
    bufs = (rows_a, rows_b)
    sems = (sem_a, sem_b)

    for t in range(NT):
        pltpu.sync_copy(idx_srcs[t].at[pl.ds(base, BPW)],
                        idx_v.at[pl.ds(t * BPW, BPW)])

    def fire(u):
        t, half = divmod(u, 2)
        table, rows_v, sem = tables[t], bufs[u % 2], sems[u % 2]

        def group_body(g, _):
            vec = idx_v[pl.ds(t * BPW + half * HB + g * L, L)]
            for j in range(L):
                i = vec[j]
                tid = lax.shift_right_logical(i, 3)
                r = i & 7
                pltpu.async_copy(table.at[tid, r], rows_v.at[g * L + j], sem)
            return ()

        lax.fori_loop(0, NGH, group_body, (), unroll=False)

    def drain_write(u):
        t, half = divmod(u, 2)
        table, rows_v, sem = tables[t], bufs[u % 2], sems[u % 2]

        def drain_body(g, _):
            for j in range(L):
                pltpu.make_async_copy(table.at[0, 0],
                                      rows_v.at[g * L + j], sem).wait()
            return ()

        lax.fori_loop(0, NGH, drain_body, (), unroll=False)
        pltpu.sync_copy(rows_v,
                        out_hbm.at[pl.ds(t * B + base + half * HB, HB)])

    fire(0)
    fire(1)
    for u in range(2, 2 * NT):
        drain_write(u - 2)
        fire(u)
    drain_write(2 * NT - 2)
    drain_write(2 * NT - 1)


def kernel(src, dst, rels, node_table, rel_table):
    node3 = node_table.reshape(125000, 8, D)
    rel3 = rel_table.reshape(125, 8, D)
    out = _gather3(src.astype(jnp.int32), dst.astype(jnp.int32),
                   rels.astype(jnp.int32), node3, rel3)
    return out.reshape(NT, B, D)


# speedup vs baseline: 2.5163x; 1.0017x over previous
import functools

import jax
import jax.numpy as jnp
from jax import lax
from jax.experimental import pallas as pl
from jax.experimental.pallas import tpu as pltpu
from jax.experimental.pallas import tpu_sc as plsc

B = 16384
D = 64
NC = 2
NS = 16
NW = NC * NS
BPW = B // NW
NT = 3
L = 16
HB = BPW // 2
NGH = HB // L

_mesh = plsc.VectorSubcoreMesh(core_axis_name="c", subcore_axis_name="s")


@functools.partial(
    pl.kernel,
    mesh=_mesh,
    out_type=jax.ShapeDtypeStruct((NT * B, D), jnp.float32),
    scratch_types=[
        pltpu.VMEM((NT * BPW,), jnp.int32),
        pltpu.VMEM((HB, D), jnp.float32),
        pltpu.VMEM((HB, D), jnp.float32),
        pltpu.SemaphoreType.DMA,
        pltpu.SemaphoreType.DMA,
    ],
)
def _gather3(src_hbm, dst_hbm, rels_hbm, node_hbm, rel_hbm, out_hbm,
             idx_v, rows_a, rows_b, sem_a, sem_b):
    wid = lax.axis_index("s") * NC + lax.axis_index("c")
    base = wid * BPW
    idx_srcs = (src_hbm, dst_hbm, rels_hbm)
    tables = (node_hbm, node_hbm, rel_hbm)
    bufs = (rows_a, rows_b)
    sems = (sem_a, sem_b)

    idx_cps = [
        pltpu.async_copy(idx_srcs[t].at[pl.ds(base, BPW)],
                         idx_v.at[pl.ds(t * BPW, BPW)], sem_a)
        for t in range(NT)
    ]
    for cp in idx_cps:
        cp.wait()

    def fire(u):
        t, half = divmod(u, 2)
        table, rows_v, sem = tables[t], bufs[u % 2], sems[u % 2]

        def group_body(g, _):
            vec = idx_v[pl.ds(t * BPW + half * HB + g * L, L)]
            for j in range(L):
                i = vec[j]
                tid = lax.shift_right_logical(i, 3)
                r = i & 7
                pltpu.async_copy(table.at[tid, r], rows_v.at[g * L + j], sem)
            return ()

        lax.fori_loop(0, NGH, group_body, (), unroll=False)

    def drain_write(u):
        t, half = divmod(u, 2)
        table, rows_v, sem = tables[t], bufs[u % 2], sems[u % 2]

        def drain_body(g, _):
            for j in range(L):
                pltpu.make_async_copy(table.at[0, 0],
                                      rows_v.at[g * L + j], sem).wait()
            return ()

        lax.fori_loop(0, NGH, drain_body, (), unroll=False)
        pltpu.sync_copy(rows_v,
                        out_hbm.at[pl.ds(t * B + base + half * HB, HB)])

    fire(0)
    fire(1)
    for u in range(2, 2 * NT):
        drain_write(u - 2)
        fire(u)
    drain_write(2 * NT - 2)
    drain_write(2 * NT - 1)


def kernel(src, dst, rels, node_table, rel_table):
    node3 = node_table.reshape(125000, 8, D)
    rel3 = rel_table.reshape(125, 8, D)
    out = _gather3(src.astype(jnp.int32), dst.astype(jnp.int32),
                   rels.astype(jnp.int32), node3, rel3)
    return out.reshape(NT, B, D)
